# Initial kernel scaffold; baseline (speedup 1.0000x reference)
import jax
import jax.numpy as jnp
from jax.experimental import pallas as pl


def kernel(x, edge_index, edge_weight, Wl, bl, Wr, br, att, gat_bias, Wxi, bxi, Whi, bhi, Wxf, bxf, Whf, bhf, Wxc, bxc, Whc, bhc, Wxo, bxo, Who, bho, wci, wcf, wco, bi, bf, bc, bo, Wout, bout):
    raise NotImplementedError("write your pallas kernel here")


# speedup vs baseline: 1.9091x; 1.9091x over previous
import jax
import jax.numpy as jnp
from jax.experimental import pallas as pl

_T = 4
_H = 128
_ROWS = 2000


def _lstm_block_kernel(x_ref, wx_ref, wh_ref, bias_ref, wci_ref, wcf_ref,
                       wco_ref, wout_ref, bout_ref, out_ref, h_ref, c_ref):
    r = h_ref.shape[0]
    wh = wh_ref[...]
    wci = wci_ref[...]
    wcf = wcf_ref[...]
    wco = wco_ref[...]

    x_all = x_ref[...].reshape(_T * r, _H)
    xw_all = (jnp.dot(x_all, wx_ref[...], preferred_element_type=jnp.float32)
              + bias_ref[...])

    h = jnp.zeros((r, _H), jnp.float32)
    c = jnp.zeros((r, _H), jnp.float32)
    for t in range(_T):
        g = xw_all[t * r:(t + 1) * r, :] + jnp.dot(
            h, wh, preferred_element_type=jnp.float32)
        gate_i = jax.nn.sigmoid(g[:, 0:_H] + wci * c)
        gate_f = jax.nn.sigmoid(g[:, _H:2 * _H] + wcf * c)
        cand = jnp.tanh(g[:, 2 * _H:3 * _H])
        c = gate_f * c + gate_i * cand
        gate_o = jax.nn.sigmoid(g[:, 3 * _H:4 * _H] + wco * c)
        h = gate_o * jnp.tanh(c)

    h_ref[...] = h
    c_ref[...] = c
    out_ref[...] = (jnp.dot(h, wout_ref[...], preferred_element_type=jnp.float32)
                    + bout_ref[...])


def kernel(x, edge_index, edge_weight, Wl, bl, Wr, br, att, gat_bias,
           Wxi, bxi, Whi, bhi, Wxf, bxf, Whf, bhf, Wxc, bxc, Whc, bhc,
           Wxo, bxo, Who, bho, wci, wcf, wco, bi, bf, bc, bo, Wout, bout):
    t_win, n, f = x.shape
    assert t_win == _T and f == _H
    rows = _ROWS
    grid = n // rows

    wx = jnp.concatenate([Wxi, Wxf, Wxc, Wxo], axis=1)
    wh = jnp.concatenate([Whi, Whf, Whc, Who], axis=1)
    bias = jnp.concatenate([bxi + bhi + bi, bxf + bhf + bf,
                            bxc + bhc + bc, bxo + bho + bo])[None, :]

    out2d, h, c = pl.pallas_call(
        _lstm_block_kernel,
        grid=(grid,),
        in_specs=[
            pl.BlockSpec((_T, rows, _H), lambda i: (0, i, 0)),
            pl.BlockSpec((_H, 4 * _H), lambda i: (0, 0)),
            pl.BlockSpec((_H, 4 * _H), lambda i: (0, 0)),
            pl.BlockSpec((1, 4 * _H), lambda i: (0, 0)),
            pl.BlockSpec((1, _H), lambda i: (0, 0)),
            pl.BlockSpec((1, _H), lambda i: (0, 0)),
            pl.BlockSpec((1, _H), lambda i: (0, 0)),
            pl.BlockSpec((_H, 1), lambda i: (0, 0)),
            pl.BlockSpec((1, 1), lambda i: (0, 0)),
        ],
        out_specs=[
            pl.BlockSpec((rows, 1), lambda i: (i, 0)),
            pl.BlockSpec((rows, _H), lambda i: (i, 0)),
            pl.BlockSpec((rows, _H), lambda i: (i, 0)),
        ],
        out_shape=[
            jax.ShapeDtypeStruct((n, 1), jnp.float32),
            jax.ShapeDtypeStruct((n, _H), jnp.float32),
            jax.ShapeDtypeStruct((n, _H), jnp.float32),
        ],
    )(x, wx, wh, bias, wci[None, :], wcf[None, :], wco[None, :],
      Wout, bout[None, :])

    return (out2d[:, 0], (h, c))


# speedup vs baseline: 1.9123x; 1.0016x over previous
import jax
import jax.numpy as jnp
from jax.experimental import pallas as pl
from jax.experimental.pallas import tpu as pltpu

_T = 4
_H = 128
_ROWS = 2000


def _lstm_block_kernel(x_ref, wx_ref, wh_ref, bias_ref, wci_ref, wcf_ref,
                       wco_ref, wout_ref, bout_ref, out_ref, h_ref, c_ref):
    r = h_ref.shape[0]
    wh = wh_ref[...]
    wci = wci_ref[...]
    wcf = wcf_ref[...]
    wco = wco_ref[...]

    x_all = x_ref[...].reshape(_T * r, _H)
    xw_all = (jnp.dot(x_all, wx_ref[...], preferred_element_type=jnp.float32)
              + bias_ref[...])

    h = jnp.zeros((r, _H), jnp.float32)
    c = jnp.zeros((r, _H), jnp.float32)
    for t in range(_T):
        g = xw_all[t * r:(t + 1) * r, :] + jnp.dot(
            h, wh, preferred_element_type=jnp.float32)
        gate_i = jax.nn.sigmoid(g[:, 0:_H] + wci * c)
        gate_f = jax.nn.sigmoid(g[:, _H:2 * _H] + wcf * c)
        cand = jnp.tanh(g[:, 2 * _H:3 * _H])
        c = gate_f * c + gate_i * cand
        gate_o = jax.nn.sigmoid(g[:, 3 * _H:4 * _H] + wco * c)
        h = gate_o * jnp.tanh(c)

    h_ref[...] = h
    c_ref[...] = c
    out_ref[...] = (jnp.dot(h, wout_ref[...], preferred_element_type=jnp.float32)
                    + bout_ref[...])


def kernel(x, edge_index, edge_weight, Wl, bl, Wr, br, att, gat_bias,
           Wxi, bxi, Whi, bhi, Wxf, bxf, Whf, bhf, Wxc, bxc, Whc, bhc,
           Wxo, bxo, Who, bho, wci, wcf, wco, bi, bf, bc, bo, Wout, bout):
    t_win, n, f = x.shape
    assert t_win == _T and f == _H
    rows = _ROWS
    grid = n // rows

    wx = jnp.concatenate([Wxi, Wxf, Wxc, Wxo], axis=1)
    wh = jnp.concatenate([Whi, Whf, Whc, Who], axis=1)
    bias = jnp.concatenate([bxi + bhi + bi, bxf + bhf + bf,
                            bxc + bhc + bc, bxo + bho + bo])[None, :]

    out2d, h, c = pl.pallas_call(
        _lstm_block_kernel,
        grid=(grid,),
        in_specs=[
            pl.BlockSpec((_T, rows, _H), lambda i: (0, i, 0)),
            pl.BlockSpec((_H, 4 * _H), lambda i: (0, 0)),
            pl.BlockSpec((_H, 4 * _H), lambda i: (0, 0)),
            pl.BlockSpec((1, 4 * _H), lambda i: (0, 0)),
            pl.BlockSpec((1, _H), lambda i: (0, 0)),
            pl.BlockSpec((1, _H), lambda i: (0, 0)),
            pl.BlockSpec((1, _H), lambda i: (0, 0)),
            pl.BlockSpec((_H, 1), lambda i: (0, 0)),
            pl.BlockSpec((1, 1), lambda i: (0, 0)),
        ],
        out_specs=[
            pl.BlockSpec((rows, 1), lambda i: (i, 0)),
            pl.BlockSpec((rows, _H), lambda i: (i, 0)),
            pl.BlockSpec((rows, _H), lambda i: (i, 0)),
        ],
        out_shape=[
            jax.ShapeDtypeStruct((n, 1), jnp.float32),
            jax.ShapeDtypeStruct((n, _H), jnp.float32),
            jax.ShapeDtypeStruct((n, _H), jnp.float32),
        ],
        compiler_params=pltpu.CompilerParams(
            dimension_semantics=("parallel",)),
    )(x, wx, wh, bias, wci[None, :], wcf[None, :], wco[None, :],
      Wout, bout[None, :])

    return (out2d[:, 0], (h, c))
